# speedup vs baseline: 1.0071x; 1.0071x over previous
import jax
import jax.numpy as jnp
from jax.experimental import pallas as pl
from jax.experimental.pallas import tpu as pltpu

N = 10000
E = 640000
AA = 21
HID = 128
NB = 64
OUT_DIM = 486
D1 = HID + AA
D2 = D1 * 2
D4 = D1 * 4
FC_IN = D4 * 2


def _gcn_conv(x, edge_index, W, b):
    n = x.shape[0]
    loop = jnp.arange(n, dtype=edge_index.dtype)
    src = jnp.concatenate([edge_index[0], loop])
    dst = jnp.concatenate([edge_index[1], loop])
    deg = jax.ops.segment_sum(jnp.ones(src.shape, x.dtype), dst, num_segments=n)
    dinv = jax.lax.rsqrt(jnp.maximum(deg, 1.0))
    norm = dinv[src] * dinv[dst]
    xw = x @ W
    msg = xw[src] * norm[:, None]
    return jax.ops.segment_sum(msg, dst, num_segments=n) + b


def _head_body(zp_ref, wfc1_ref, bfc1_ref, gamma_ref, beta_ref, wfc2_ref, bfc2_ref, out_ref):
    h = jnp.dot(zp_ref[...], wfc1_ref[...], preferred_element_type=jnp.float32) + bfc1_ref[...]
    mu = jnp.mean(h, axis=0, keepdims=True)
    var = jnp.mean((h - mu) ** 2, axis=0, keepdims=True)
    h = (h - mu) * jax.lax.rsqrt(var + 1e-5) * gamma_ref[...] + beta_ref[...]
    h = jnp.maximum(h, 0.0)
    o = jnp.dot(h, wfc2_ref[...], preferred_element_type=jnp.float32) + bfc2_ref[...]
    out_ref[...] = jax.nn.sigmoid(o)


def kernel(prot_x, prot_edge_index, prot_edge_index_replace, prot_batch, W_fl1, b_fl1, W_fl2, b_fl2, W_p1, b_p1, W_p2, b_p2, W_a1, b_a1, W_a2, b_a2, W_p3, b_p3, W_fc1, b_fc1, bn_gamma, bn_beta, W_fc2, b_fc2):
    f1 = jax.nn.relu(prot_x[:, AA:] @ W_fl1 + b_fl1)
    f2 = jax.nn.relu(prot_x[:, :AA] @ W_fl2 + b_fl2)
    feat = jnp.concatenate([f2, f1], axis=1)
    x = jax.nn.relu(_gcn_conv(feat, prot_edge_index, W_p1, b_p1))
    x = jax.nn.relu(_gcn_conv(x, prot_edge_index, W_p2, b_p2))
    y = jax.nn.relu(_gcn_conv(feat, prot_edge_index_replace, W_a1, b_a1))
    y = jax.nn.relu(_gcn_conv(y, prot_edge_index_replace, W_a2, b_a2))
    zc = jnp.concatenate([x, y], axis=1)
    z = jax.nn.relu(_gcn_conv(zc, prot_edge_index, W_p3, b_p3))
    counts = jax.ops.segment_sum(jnp.ones((z.shape[0],), z.dtype), prot_batch, num_segments=NB)
    zmean = jax.ops.segment_sum(z, prot_batch, num_segments=NB) / jnp.maximum(counts, 1.0)[:, None]
    zmax = jax.ops.segment_max(z, prot_batch, num_segments=NB)
    zmax = jnp.where(counts[:, None] > 0, zmax, 0.0)
    zp = jnp.concatenate([zmean, zmax], axis=1)
    out = pl.pallas_call(
        _head_body,
        out_shape=jax.ShapeDtypeStruct((NB, OUT_DIM), jnp.float32),
    )(zp, W_fc1, b_fc1, bn_gamma, bn_beta, W_fc2, b_fc2)
    return out


# speedup vs baseline: 5.5881x; 5.5485x over previous
import jax
import jax.numpy as jnp
from jax import lax
from jax.experimental import pallas as pl
from jax.experimental.pallas import tpu as pltpu
from jax.experimental.pallas import tpu_sc as plsc

N = 10000
E = 640000
AA = 21
HID = 128
NB = 64
OUT_DIM = 486
D1 = HID + AA
D2 = D1 * 2
D4 = D1 * 4

NP = 10240
RPT = 640
F = 128
NT = 16
EB = 128
NBATCH = 320
KW = 16
NW = NBATCH // KW
EP = NT * NBATCH * EB
BQ = NP // EB

_f32 = jnp.float32
_i32 = jnp.int32

_MESH = plsc.VectorSubcoreMesh(core_axis_name="c", subcore_axis_name="s")


def _deg_body(dstp, dstr, batch2r, erows, z2, degp, degr, counts,
              dacc, cacc, didx, bidx, eones):
    core = lax.axis_index("c")
    sid = lax.axis_index("s")
    r0 = sid * RPT
    pltpu.sync_copy(erows, eones)
    pltpu.sync_copy(z2.at[pl.ds(r0, RPT)], dacc.at[pl.ds(r0, RPT)])
    pltpu.sync_copy(z2.at[pl.ds(sid * 8, 8)], cacc.at[pl.ds(sid * 8, 8)])

    plsc.subcore_barrier()

    def eb(b, carry):
        @pl.when(core == 0)
        def _():
            pltpu.sync_copy(dstp.at[sid].at[b].at[0], didx)

        @pl.when(core == 1)
        def _():
            pltpu.sync_copy(dstr.at[sid].at[b].at[0], didx)

        pltpu.sync_copy(eones, dacc.at[didx], add=True)
        return carry

    lax.fori_loop(0, NBATCH, eb, 0)

    @pl.when(core == 0)
    def _():

        def cb(b, carry):
            pltpu.sync_copy(batch2r.at[sid].at[b].at[0], bidx)
            pltpu.sync_copy(eones, cacc.at[bidx], add=True)
            return carry

        lax.fori_loop(0, BQ // NT, cb, 0)

    plsc.subcore_barrier()

    @pl.when(core == 0)
    def _():
        pltpu.sync_copy(dacc.at[pl.ds(r0, RPT)], degp.at[pl.ds(r0, RPT)])
        pltpu.sync_copy(cacc.at[pl.ds(sid * 8, 8)],
                        counts.at[pl.ds(sid * 8, 8)])

    @pl.when(core == 1)
    def _():
        pltpu.sync_copy(dacc.at[pl.ds(r0, RPT)], degr.at[pl.ds(r0, RPT)])


def _sc_deg(dstp, dstr, batch2r, erows, z2):
    return pl.kernel(
        _deg_body,
        out_type=(
            jax.ShapeDtypeStruct((NP, F), _f32),
            jax.ShapeDtypeStruct((NP, F), _f32),
            jax.ShapeDtypeStruct((EB, F), _f32),
        ),
        mesh=_MESH,
        scratch_types=[
            pltpu.VMEM_SHARED((NP, F), _f32),
            pltpu.VMEM_SHARED((EB, F), _f32),
            pltpu.VMEM((EB,), _i32),
            pltpu.VMEM((EB,), _i32),
            pltpu.VMEM((EB, F), _f32),
        ],
    )(dstp, dstr, batch2r, erows, z2)


def _conv_body(nc, parity, xwc, srcp, dstp, z2, out, acc, sidx, didx, rows):
    core = lax.axis_index("c")
    sid = lax.axis_index("s")
    r0 = sid * RPT
    for ch in range(nc):
        owner = (ch + parity) % 2

        @pl.when(core == owner)
        def _(ch=ch):
            pltpu.sync_copy(z2.at[pl.ds(r0, RPT)], acc.at[pl.ds(r0, RPT)])
            plsc.subcore_barrier()

            def eb(b, carry):
                pltpu.sync_copy(srcp.at[sid].at[b].at[0], sidx)
                pltpu.sync_copy(dstp.at[sid].at[b].at[0], didx)
                pltpu.sync_copy(xwc.at[ch].at[sidx], rows)
                pltpu.sync_copy(rows, acc.at[didx], add=True)
                return carry

            lax.fori_loop(0, NBATCH, eb, 0)
            plsc.subcore_barrier()
            pltpu.sync_copy(acc.at[pl.ds(r0, RPT)],
                            out.at[ch].at[pl.ds(r0, RPT)])


def _sc_conv(nc, parity, xwc, srcp, dstp, z2):
    def body(xwc, srcp, dstp, z2, out, acc, sidx, didx, rows):
        _conv_body(nc, parity, xwc, srcp, dstp, z2, out, acc, sidx, didx, rows)

    return pl.kernel(
        body,
        out_type=jax.ShapeDtypeStruct((nc, NP, F), _f32),
        mesh=_MESH,
        scratch_types=[
            pltpu.VMEM_SHARED((NP, F), _f32),
            pltpu.VMEM((EB,), _i32),
            pltpu.VMEM((EB,), _i32),
            pltpu.VMEM((EB, F), _f32),
        ],
    )(xwc, srcp, dstp, z2)


def _pool_body(acc5, xw5, dinv, batchp, b3p, z2, psum, pmax,
               abuf, xbuf, sacc, macc, bvec, dbuf, bibuf):
    core = lax.axis_index("c")
    sid = lax.axis_index("s")
    r0 = sid * RPT
    pltpu.sync_copy(dinv.at[pl.ds(r0, RPT)], dbuf)
    pltpu.sync_copy(batchp.at[pl.ds(r0, RPT)], bibuf)

    for ch in range(5):
        owner = ch % 2

        @pl.when(core == owner)
        def _(ch=ch):
            pltpu.sync_copy(b3p.at[ch], bvec)
            pltpu.sync_copy(z2.at[pl.ds(0, BQ)], sacc)
            pltpu.sync_copy(z2.at[pl.ds(0, BQ)], macc)

            def blk_body(blk, carry2):
                pltpu.sync_copy(acc5.at[ch].at[pl.ds(r0 + blk * EB, EB)], abuf)
                pltpu.sync_copy(xw5.at[ch].at[pl.ds(r0 + blk * EB, EB)], xbuf)

                def grpbody(g, carry3):
                    b16 = bibuf[pl.ds(blk * EB + g * 16, 16)]
                    d16 = dbuf[pl.ds(blk * EB + g * 16, 16)]
                    for k in range(16):
                        i = g * 16 + k
                        b = b16[k]
                        dv = d16[k]
                        for j in range(F // 16):
                            sl = pl.ds(j * 16, 16)
                            v = (abuf[i, sl] + xbuf[i, sl]) * dv + bvec[0, sl]
                            v = jnp.maximum(v, 0.0)
                            sacc[b, sl] = sacc[b, sl] + v
                            macc[b, sl] = jnp.maximum(macc[b, sl], v)
                    return carry3

                lax.fori_loop(0, EB // 16, grpbody, 0)
                return carry2

            lax.fori_loop(0, RPT // EB, blk_body, 0)
            pltpu.sync_copy(sacc.at[pl.ds(0, NB)], psum.at[ch].at[sid])
            pltpu.sync_copy(macc.at[pl.ds(0, NB)], pmax.at[ch].at[sid])


def _sc_pool(acc5, xw5, dinv, batchp, b3p, z2):
    return pl.kernel(
        _pool_body,
        out_type=(
            jax.ShapeDtypeStruct((5, NT, NB, F), _f32),
            jax.ShapeDtypeStruct((5, NT, NB, F), _f32),
        ),
        mesh=_MESH,
        scratch_types=[
            pltpu.VMEM((EB, F), _f32),
            pltpu.VMEM((EB, F), _f32),
            pltpu.VMEM((BQ, F), _f32),
            pltpu.VMEM((BQ, F), _f32),
            pltpu.VMEM((1, 128), _f32),
            pltpu.VMEM((RPT,), _f32),
            pltpu.VMEM((RPT,), _i32),
        ],
    )(acc5, xw5, dinv, batchp, b3p, z2)


def _prep_body(x_ref, wbig_ref, bbig_ref, degp_ref, degr_ref,
               feat_ref, dinvp_ref, dinvr_ref):
    feat_ref[...] = jnp.maximum(
        jnp.dot(x_ref[...], wbig_ref[...], preferred_element_type=_f32)
        + bbig_ref[...], 0.0)
    mask = (lax.broadcasted_iota(_i32, (NP, 1), 0) < N).astype(_f32)
    dinvp_ref[...] = lax.rsqrt(degp_ref[...] + 1.0) * mask
    dinvr_ref[...] = lax.rsqrt(degr_ref[...] + 1.0) * mask


def _tc_prep(xpad, wbig, bbig, degp, degr):
    return pl.pallas_call(
        _prep_body,
        out_shape=(
            jax.ShapeDtypeStruct((NP, 256), _f32),
            jax.ShapeDtypeStruct((NP, 1), _f32),
            jax.ShapeDtypeStruct((NP, 1), _f32),
        ),
    )(xpad, wbig, bbig, degp, degr)


def _mm_body(nci, x_ref, w_ref, dinv_ref, out_ref):
    ci = pl.program_id(1)
    h = jnp.dot(x_ref[0], w_ref[0, 0], preferred_element_type=_f32)

    @pl.when(ci == 0)
    def _():
        out_ref[0] = h

    @pl.when(ci > 0)
    def _():
        out_ref[0] = out_ref[0] + h

    @pl.when(ci == nci - 1)
    def _():
        out_ref[0] = dinv_ref[...] * out_ref[0]


def _tc_mm(x3, w4, dinv):
    nci, nco = w4.shape[0], w4.shape[1]
    import functools
    return pl.pallas_call(
        functools.partial(_mm_body, nci),
        grid=(nco, nci),
        in_specs=[
            pl.BlockSpec((1, NP, F), lambda c, ci: (ci, 0, 0)),
            pl.BlockSpec((1, 1, F, F), lambda c, ci: (ci, c, 0, 0)),
            pl.BlockSpec((NP, 1), lambda c, ci: (0, 0)),
        ],
        out_specs=pl.BlockSpec((1, NP, F), lambda c, ci: (c, 0, 0)),
        out_shape=jax.ShapeDtypeStruct((nco, NP, F), _f32),
    )(x3, w4, dinv)


def _ew_body(acc_ref, xw_ref, dinv_ref, b_ref, out_ref):
    out_ref[0] = jnp.maximum(
        dinv_ref[...] * (acc_ref[0] + xw_ref[0]) + b_ref[0], 0.0)


def _tc_ew(accc, xwc, dinv, bpad):
    nc = accc.shape[0]
    return pl.pallas_call(
        _ew_body,
        grid=(nc,),
        in_specs=[
            pl.BlockSpec((1, NP, F), lambda c: (c, 0, 0)),
            pl.BlockSpec((1, NP, F), lambda c: (c, 0, 0)),
            pl.BlockSpec((NP, 1), lambda c: (0, 0)),
            pl.BlockSpec((1, 1, F), lambda c: (c, 0, 0)),
        ],
        out_specs=pl.BlockSpec((1, NP, F), lambda c: (c, 0, 0)),
        out_shape=jax.ShapeDtypeStruct((nc, NP, F), _f32),
    )(accc, xwc, dinv, bpad)


def _comb_body(psum_ref, pmax_ref, cnt_ref, zmean_ref, zmax_ref):
    denom = jnp.maximum(cnt_ref[...], 1.0)
    zmean_ref[0] = jnp.sum(psum_ref[0], axis=0) / denom
    zmax_ref[0] = jnp.max(pmax_ref[0], axis=0)


def _tc_comb(psum, pmax, cnt):
    return pl.pallas_call(
        _comb_body,
        grid=(5,),
        in_specs=[
            pl.BlockSpec((1, NT, NB, F), lambda c: (c, 0, 0, 0)),
            pl.BlockSpec((1, NT, NB, F), lambda c: (c, 0, 0, 0)),
            pl.BlockSpec((NB, 1), lambda c: (0, 0)),
        ],
        out_specs=(
            pl.BlockSpec((1, NB, F), lambda c: (c, 0, 0)),
            pl.BlockSpec((1, NB, F), lambda c: (c, 0, 0)),
        ),
        out_shape=(
            jax.ShapeDtypeStruct((5, NB, F), _f32),
            jax.ShapeDtypeStruct((5, NB, F), _f32),
        ),
    )(psum, pmax, cnt)


def _head_body(zp_ref, wfc1_ref, bfc1_ref, gamma_ref, beta_ref, wfc2_ref,
               bfc2_ref, out_ref):
    h = jnp.dot(zp_ref[...], wfc1_ref[...], preferred_element_type=_f32) \
        + bfc1_ref[...]
    mu = jnp.mean(h, axis=0, keepdims=True)
    var = jnp.mean((h - mu) ** 2, axis=0, keepdims=True)
    h = (h - mu) * lax.rsqrt(var + 1e-5) * gamma_ref[...] + beta_ref[...]
    h = jnp.maximum(h, 0.0)
    o = jnp.dot(h, wfc2_ref[...], preferred_element_type=_f32) + bfc2_ref[...]
    out_ref[...] = jax.nn.sigmoid(o)


def _tc_head(zp, wfc1, bfc1, gamma, beta, wfc2, bfc2):
    return pl.pallas_call(
        _head_body,
        out_shape=jax.ShapeDtypeStruct((NB, OUT_DIM), _f32),
    )(zp, wfc1, bfc1, gamma, beta, wfc2, bfc2)


def _pad2(w, rows, cols):
    return jnp.zeros((rows, cols), _f32).at[:w.shape[0], :w.shape[1]].set(w)


def _w4(w2d):
    kp, dp = w2d.shape
    return w2d.reshape(kp // F, F, dp // F, F).transpose(0, 2, 1, 3)


def _pad_edges(ei):
    npad = EP - E
    psrc = (jnp.arange(npad, dtype=_i32) % 8192)
    pdst = N + (jnp.arange(npad, dtype=_i32) % (NP - N))
    src = jnp.concatenate([ei[0], psrc]).reshape(NT, NBATCH, 1, EB)
    dst = jnp.concatenate([ei[1], pdst]).reshape(NT, NBATCH, 1, EB)
    return src, dst


def kernel(prot_x, prot_edge_index, prot_edge_index_replace, prot_batch,
           W_fl1, b_fl1, W_fl2, b_fl2, W_p1, b_p1, W_p2, b_p2, W_a1, b_a1,
           W_a2, b_a2, W_p3, b_p3, W_fc1, b_fc1, bn_gamma, bn_beta,
           W_fc2, b_fc2):
    srcp, dstp = _pad_edges(prot_edge_index)
    srcr, dstr = _pad_edges(prot_edge_index_replace)
    batch_pad = jnp.concatenate(
        [prot_batch, NB + (jnp.arange(NP - N, dtype=_i32) % (BQ - NB))])
    batch2r = batch_pad.reshape(NT, BQ // NT, 1, EB)
    erows = jnp.zeros((EB, F), _f32).at[:, 0].set(1.0)
    xpad = _pad2(prot_x, NP, 256)
    wbig = jnp.zeros((256, 256), _f32)
    wbig = wbig.at[:AA, :AA].set(W_fl2).at[AA:D1, AA:D1].set(W_fl1)
    bbig = jnp.zeros((1, 256), _f32).at[0, :AA].set(b_fl2).at[0, AA:D1].set(b_fl1)
    wp1 = _w4(_pad2(W_p1, 256, 256))
    wp2 = _w4(_pad2(W_p2, 256, 384))
    wa1 = _w4(_pad2(W_a1, 256, 256))
    wa2 = _w4(_pad2(W_a2, 256, 384))
    wp3 = jnp.zeros((768, 640), _f32)
    wp3 = wp3.at[:D2, :D4].set(W_p3[:D2]).at[384:384 + D2, :D4].set(W_p3[D2:])
    wp3 = _w4(wp3)
    bp1 = _pad2(b_p1[None, :], 1, 256).reshape(2, 1, F)
    bp2 = _pad2(b_p2[None, :], 1, 384).reshape(3, 1, F)
    ba1 = _pad2(b_a1[None, :], 1, 256).reshape(2, 1, F)
    ba2 = _pad2(b_a2[None, :], 1, 384).reshape(3, 1, F)
    bp3 = _pad2(b_p3[None, :], 1, 640).reshape(5, 1, F)
    wfc1 = jnp.zeros((1280, 1024), _f32)
    wfc1 = wfc1.at[:D4, :].set(W_fc1[:D4]).at[640:640 + D4, :].set(W_fc1[D4:])
    z2 = jnp.zeros((NP, F), _f32)

    degp, degr, counts = _sc_deg(dstp, dstr, batch2r, erows, z2)
    cnt = counts[:NB, :1]

    feat2d, dinvp, dinvr = _tc_prep(xpad, wbig, bbig,
                                    degp[:, :1], degr[:, :1])
    feat = feat2d.reshape(NP, 2, F).transpose(1, 0, 2)

    xw1 = _tc_mm(feat, wp1, dinvp)
    acc1 = _sc_conv(2, 0, xw1, srcp, dstp, z2)
    x1 = _tc_ew(acc1, xw1, dinvp, bp1)
    xw2 = _tc_mm(x1, wp2, dinvp)
    acc2 = _sc_conv(3, 0, xw2, srcp, dstp, z2)
    x2 = _tc_ew(acc2, xw2, dinvp, bp2)

    xw3 = _tc_mm(feat, wa1, dinvr)
    acc3 = _sc_conv(2, 1, xw3, srcr, dstr, z2)
    y1 = _tc_ew(acc3, xw3, dinvr, ba1)
    xw4 = _tc_mm(y1, wa2, dinvr)
    acc4 = _sc_conv(3, 1, xw4, srcr, dstr, z2)
    y2 = _tc_ew(acc4, xw4, dinvr, ba2)

    zc = jnp.concatenate([x2, y2], axis=0)
    xw5 = _tc_mm(zc, wp3, dinvp)
    acc5 = _sc_conv(5, 0, xw5, srcp, dstp, z2)

    psum, pmax = _sc_pool(acc5, xw5, dinvp.reshape(NP), batch_pad, bp3, z2)
    zmean, zmax = _tc_comb(psum, pmax, cnt)
    zp = jnp.concatenate([zmean.transpose(1, 0, 2).reshape(NB, 640),
                          zmax.transpose(1, 0, 2).reshape(NB, 640)], axis=1)

    out = _tc_head(zp, wfc1, b_fc1.reshape(1, 1024), bn_gamma.reshape(1, 1024),
                   bn_beta.reshape(1, 1024), W_fc2, b_fc2.reshape(1, OUT_DIM))
    return out


# speedup vs baseline: 10.0655x; 1.8012x over previous
import jax
import jax.numpy as jnp
from jax import lax
from jax.experimental import pallas as pl
from jax.experimental.pallas import tpu as pltpu
from jax.experimental.pallas import tpu_sc as plsc

N = 10000
E = 640000
AA = 21
HID = 128
NB = 64
OUT_DIM = 486
D1 = HID + AA
D2 = D1 * 2
D4 = D1 * 4

NP = 10240
RPT = 640
F = 128
NT = 16
EB = 128
NBATCH = 320
G = 16
NW = NBATCH // G
EP = NT * NBATCH * EB
BQ = NP // EB

_f32 = jnp.float32
_i32 = jnp.int32

_MESH = plsc.VectorSubcoreMesh(core_axis_name="c", subcore_axis_name="s")


def _deg_body(sep, ser, batch2r, erows, z2, degp, degr, counts,
              dacc, cacc, eidx, bidx, eones, ss0):
    core = lax.axis_index("c")
    sid = lax.axis_index("s")
    r0 = sid * RPT
    pltpu.sync_copy(erows, eones)
    pltpu.sync_copy(z2.at[pl.ds(r0, RPT)], dacc.at[pl.ds(r0, RPT)])
    pltpu.sync_copy(z2.at[pl.ds(sid * 8, 8)], cacc.at[pl.ds(sid * 8, 8)])

    plsc.subcore_barrier()

    def wloop(w, carry):
        @pl.when(core == 0)
        def _():
            pltpu.sync_copy(sep.at[sid].at[w], eidx)

        @pl.when(core == 1)
        def _():
            pltpu.sync_copy(ser.at[sid].at[w], eidx)

        for g in range(G):
            pltpu.async_copy(eones, dacc.at[eidx.at[g].at[1].at[0]], ss0,
                             add=True)
        for g in range(G):
            pltpu.make_async_copy(eones, dacc.at[eidx.at[g].at[1].at[0]],
                                  ss0).wait()
        return carry

    lax.fori_loop(0, NW, wloop, 0)

    @pl.when(core == 0)
    def _():

        def cb(b, carry):
            pltpu.sync_copy(batch2r.at[sid].at[b].at[0], bidx)
            pltpu.sync_copy(eones, cacc.at[bidx], add=True)
            return carry

        lax.fori_loop(0, BQ // NT, cb, 0)

    plsc.subcore_barrier()

    @pl.when(core == 0)
    def _():
        pltpu.sync_copy(dacc.at[pl.ds(r0, RPT)], degp.at[pl.ds(r0, RPT)])
        pltpu.sync_copy(cacc.at[pl.ds(sid * 8, 8)],
                        counts.at[pl.ds(sid * 8, 8)])

    @pl.when(core == 1)
    def _():
        pltpu.sync_copy(dacc.at[pl.ds(r0, RPT)], degr.at[pl.ds(r0, RPT)])


def _sc_deg(sep, ser, batch2r, erows, z2):
    return pl.kernel(
        _deg_body,
        out_type=(
            jax.ShapeDtypeStruct((NP, F), _f32),
            jax.ShapeDtypeStruct((NP, F), _f32),
            jax.ShapeDtypeStruct((EB, F), _f32),
        ),
        mesh=_MESH,
        scratch_types=[
            pltpu.VMEM_SHARED((NP, F), _f32),
            pltpu.VMEM_SHARED((EB, F), _f32),
            pltpu.VMEM((G, 2, 1, EB), _i32),
            pltpu.VMEM((EB,), _i32),
            pltpu.VMEM((EB, F), _f32),
            pltpu.SemaphoreType.DMA,
        ],
    )(sep, ser, batch2r, erows, z2)


def _conv_body(nc, parity, xwc, se, z2, out, acc, eidx, rows,
               gs0, gs1, ss0, ss1):
    core = lax.axis_index("c")
    sid = lax.axis_index("s")
    r0 = sid * RPT
    GS = (gs0, gs1)
    SS = (ss0, ss1)
    for ch in range(nc):
        owner = (ch + parity) % 2

        @pl.when(core == owner)
        def _(ch=ch):
            pltpu.sync_copy(z2.at[pl.ds(r0, RPT)], acc.at[pl.ds(r0, RPT)])
            plsc.subcore_barrier()
            tbl = xwc.at[ch]

            def wloop(w, carry):
                pltpu.sync_copy(se.at[sid].at[w], eidx)

                def sref(g):
                    return eidx.at[g].at[0].at[0]

                def dref(g):
                    return eidx.at[g].at[1].at[0]

                R = (rows.at[0], rows.at[1])
                pltpu.async_copy(tbl.at[sref(0)], R[0], GS[0])
                for g in range(G):
                    A = g % 2
                    B = 1 - A
                    pltpu.make_async_copy(tbl.at[sref(g)], R[A], GS[A]).wait()
                    if g < G - 1:
                        if g >= 1:
                            pltpu.make_async_copy(
                                R[B], acc.at[dref(g - 1)], SS[B]).wait()
                        pltpu.async_copy(tbl.at[sref(g + 1)], R[B], GS[B])
                    pltpu.async_copy(R[A], acc.at[dref(g)], SS[A], add=True)
                pltpu.make_async_copy(R[0], acc.at[dref(G - 2)], SS[0]).wait()
                pltpu.make_async_copy(R[1], acc.at[dref(G - 1)], SS[1]).wait()
                return carry

            lax.fori_loop(0, NW, wloop, 0)
            plsc.subcore_barrier()
            pltpu.sync_copy(acc.at[pl.ds(r0, RPT)],
                            out.at[ch].at[pl.ds(r0, RPT)])


def _sc_conv(nc, parity, xwc, se, z2):
    def body(xwc, se, z2, out, acc, eidx, rows, gs0, gs1, ss0, ss1):
        _conv_body(nc, parity, xwc, se, z2, out, acc, eidx, rows,
                   gs0, gs1, ss0, ss1)

    return pl.kernel(
        body,
        out_type=jax.ShapeDtypeStruct((nc, NP, F), _f32),
        mesh=_MESH,
        scratch_types=[
            pltpu.VMEM_SHARED((NP, F), _f32),
            pltpu.VMEM((G, 2, 1, EB), _i32),
            pltpu.VMEM((2, EB, F), _f32),
            pltpu.SemaphoreType.DMA,
            pltpu.SemaphoreType.DMA,
            pltpu.SemaphoreType.DMA,
            pltpu.SemaphoreType.DMA,
        ],
    )(xwc, se, z2)


def _pool_body(acc5, xw5, dinv, batchp, b3p, z2, psum, pmax,
               abuf, xbuf, sacc, macc, bvec, dbuf, bibuf):
    core = lax.axis_index("c")
    sid = lax.axis_index("s")
    r0 = sid * RPT
    pltpu.sync_copy(dinv.at[pl.ds(r0, RPT)], dbuf)
    pltpu.sync_copy(batchp.at[pl.ds(r0, RPT)], bibuf)

    for ch in range(5):
        owner = ch % 2

        @pl.when(core == owner)
        def _(ch=ch):
            pltpu.sync_copy(b3p.at[ch], bvec)
            pltpu.sync_copy(z2.at[pl.ds(0, BQ)], sacc)
            pltpu.sync_copy(z2.at[pl.ds(0, BQ)], macc)

            def blk_body(blk, carry2):
                pltpu.sync_copy(acc5.at[ch].at[pl.ds(r0 + blk * EB, EB)], abuf)
                pltpu.sync_copy(xw5.at[ch].at[pl.ds(r0 + blk * EB, EB)], xbuf)

                def grpbody(g, carry3):
                    b16 = bibuf[pl.ds(blk * EB + g * 16, 16)]
                    d16 = dbuf[pl.ds(blk * EB + g * 16, 16)]
                    for k in range(16):
                        i = g * 16 + k
                        b = b16[k]
                        dv = d16[k]
                        for j in range(F // 16):
                            sl = pl.ds(j * 16, 16)
                            v = (abuf[i, sl] + xbuf[i, sl]) * dv + bvec[0, sl]
                            v = jnp.maximum(v, 0.0)
                            sacc[b, sl] = sacc[b, sl] + v
                            macc[b, sl] = jnp.maximum(macc[b, sl], v)
                    return carry3

                lax.fori_loop(0, EB // 16, grpbody, 0)
                return carry2

            lax.fori_loop(0, RPT // EB, blk_body, 0)
            pltpu.sync_copy(sacc.at[pl.ds(0, NB)], psum.at[ch].at[sid])
            pltpu.sync_copy(macc.at[pl.ds(0, NB)], pmax.at[ch].at[sid])


def _sc_pool(acc5, xw5, dinv, batchp, b3p, z2):
    return pl.kernel(
        _pool_body,
        out_type=(
            jax.ShapeDtypeStruct((5, NT, NB, F), _f32),
            jax.ShapeDtypeStruct((5, NT, NB, F), _f32),
        ),
        mesh=_MESH,
        scratch_types=[
            pltpu.VMEM((EB, F), _f32),
            pltpu.VMEM((EB, F), _f32),
            pltpu.VMEM((BQ, F), _f32),
            pltpu.VMEM((BQ, F), _f32),
            pltpu.VMEM((1, 128), _f32),
            pltpu.VMEM((RPT,), _f32),
            pltpu.VMEM((RPT,), _i32),
        ],
    )(acc5, xw5, dinv, batchp, b3p, z2)


def _prep_body(x_ref, wbig_ref, bbig_ref, degp_ref, degr_ref,
               feat_ref, dinvp_ref, dinvr_ref):
    feat_ref[...] = jnp.maximum(
        jnp.dot(x_ref[...], wbig_ref[...], preferred_element_type=_f32)
        + bbig_ref[...], 0.0)
    mask = (lax.broadcasted_iota(_i32, (NP, 1), 0) < N).astype(_f32)
    dinvp_ref[...] = lax.rsqrt(degp_ref[...] + 1.0) * mask
    dinvr_ref[...] = lax.rsqrt(degr_ref[...] + 1.0) * mask


def _tc_prep(xpad, wbig, bbig, degp, degr):
    return pl.pallas_call(
        _prep_body,
        out_shape=(
            jax.ShapeDtypeStruct((NP, 256), _f32),
            jax.ShapeDtypeStruct((NP, 1), _f32),
            jax.ShapeDtypeStruct((NP, 1), _f32),
        ),
    )(xpad, wbig, bbig, degp, degr)


def _mm_body(nci, x_ref, w_ref, dinv_ref, out_ref):
    ci = pl.program_id(1)
    h = jnp.dot(x_ref[0], w_ref[0, 0], preferred_element_type=_f32)

    @pl.when(ci == 0)
    def _():
        out_ref[0] = h

    @pl.when(ci > 0)
    def _():
        out_ref[0] = out_ref[0] + h

    @pl.when(ci == nci - 1)
    def _():
        out_ref[0] = dinv_ref[...] * out_ref[0]


def _tc_mm(x3, w4, dinv):
    nci, nco = w4.shape[0], w4.shape[1]
    import functools
    return pl.pallas_call(
        functools.partial(_mm_body, nci),
        grid=(nco, nci),
        in_specs=[
            pl.BlockSpec((1, NP, F), lambda c, ci: (ci, 0, 0)),
            pl.BlockSpec((1, 1, F, F), lambda c, ci: (ci, c, 0, 0)),
            pl.BlockSpec((NP, 1), lambda c, ci: (0, 0)),
        ],
        out_specs=pl.BlockSpec((1, NP, F), lambda c, ci: (c, 0, 0)),
        out_shape=jax.ShapeDtypeStruct((nco, NP, F), _f32),
    )(x3, w4, dinv)


def _ew_body(acc_ref, xw_ref, dinv_ref, b_ref, out_ref):
    out_ref[0] = jnp.maximum(
        dinv_ref[...] * (acc_ref[0] + xw_ref[0]) + b_ref[0], 0.0)


def _tc_ew(accc, xwc, dinv, bpad):
    nc = accc.shape[0]
    return pl.pallas_call(
        _ew_body,
        grid=(nc,),
        in_specs=[
            pl.BlockSpec((1, NP, F), lambda c: (c, 0, 0)),
            pl.BlockSpec((1, NP, F), lambda c: (c, 0, 0)),
            pl.BlockSpec((NP, 1), lambda c: (0, 0)),
            pl.BlockSpec((1, 1, F), lambda c: (c, 0, 0)),
        ],
        out_specs=pl.BlockSpec((1, NP, F), lambda c: (c, 0, 0)),
        out_shape=jax.ShapeDtypeStruct((nc, NP, F), _f32),
    )(accc, xwc, dinv, bpad)


def _comb_body(psum_ref, pmax_ref, cnt_ref, zmean_ref, zmax_ref):
    denom = jnp.maximum(cnt_ref[...], 1.0)
    zmean_ref[0] = jnp.sum(psum_ref[0], axis=0) / denom
    zmax_ref[0] = jnp.max(pmax_ref[0], axis=0)


def _tc_comb(psum, pmax, cnt):
    return pl.pallas_call(
        _comb_body,
        grid=(5,),
        in_specs=[
            pl.BlockSpec((1, NT, NB, F), lambda c: (c, 0, 0, 0)),
            pl.BlockSpec((1, NT, NB, F), lambda c: (c, 0, 0, 0)),
            pl.BlockSpec((NB, 1), lambda c: (0, 0)),
        ],
        out_specs=(
            pl.BlockSpec((1, NB, F), lambda c: (c, 0, 0)),
            pl.BlockSpec((1, NB, F), lambda c: (c, 0, 0)),
        ),
        out_shape=(
            jax.ShapeDtypeStruct((5, NB, F), _f32),
            jax.ShapeDtypeStruct((5, NB, F), _f32),
        ),
    )(psum, pmax, cnt)


def _head_body(zp_ref, wfc1_ref, bfc1_ref, gamma_ref, beta_ref, wfc2_ref,
               bfc2_ref, out_ref):
    h = jnp.dot(zp_ref[...], wfc1_ref[...], preferred_element_type=_f32) \
        + bfc1_ref[...]
    mu = jnp.mean(h, axis=0, keepdims=True)
    var = jnp.mean((h - mu) ** 2, axis=0, keepdims=True)
    h = (h - mu) * lax.rsqrt(var + 1e-5) * gamma_ref[...] + beta_ref[...]
    h = jnp.maximum(h, 0.0)
    o = jnp.dot(h, wfc2_ref[...], preferred_element_type=_f32) + bfc2_ref[...]
    out_ref[...] = jax.nn.sigmoid(o)


def _tc_head(zp, wfc1, bfc1, gamma, beta, wfc2, bfc2):
    return pl.pallas_call(
        _head_body,
        out_shape=jax.ShapeDtypeStruct((NB, OUT_DIM), _f32),
    )(zp, wfc1, bfc1, gamma, beta, wfc2, bfc2)


def _pad2(w, rows, cols):
    return jnp.zeros((rows, cols), _f32).at[:w.shape[0], :w.shape[1]].set(w)


def _w4(w2d):
    kp, dp = w2d.shape
    return w2d.reshape(kp // F, F, dp // F, F).transpose(0, 2, 1, 3)


def _pad_edges(ei):
    npad = EP - E
    psrc = (jnp.arange(npad, dtype=_i32) % 8192)
    pdst = N + (jnp.arange(npad, dtype=_i32) % (NP - N))
    src = jnp.concatenate([ei[0], psrc]).reshape(NT, NBATCH, EB)
    dst = jnp.concatenate([ei[1], pdst]).reshape(NT, NBATCH, EB)
    se = jnp.stack([src, dst], axis=2)
    return se.reshape(NT, NW, G, 2, 1, EB)


def kernel(prot_x, prot_edge_index, prot_edge_index_replace, prot_batch,
           W_fl1, b_fl1, W_fl2, b_fl2, W_p1, b_p1, W_p2, b_p2, W_a1, b_a1,
           W_a2, b_a2, W_p3, b_p3, W_fc1, b_fc1, bn_gamma, bn_beta,
           W_fc2, b_fc2):
    sep = _pad_edges(prot_edge_index)
    ser = _pad_edges(prot_edge_index_replace)
    batch_pad = jnp.concatenate(
        [prot_batch, NB + (jnp.arange(NP - N, dtype=_i32) % (BQ - NB))])
    batch2r = batch_pad.reshape(NT, BQ // NT, 1, EB)
    erows = jnp.zeros((EB, F), _f32).at[:, 0].set(1.0)
    xpad = _pad2(prot_x, NP, 256)
    wbig = jnp.zeros((256, 256), _f32)
    wbig = wbig.at[:AA, :AA].set(W_fl2).at[AA:D1, AA:D1].set(W_fl1)
    bbig = jnp.zeros((1, 256), _f32).at[0, :AA].set(b_fl2).at[0, AA:D1].set(b_fl1)
    wp1 = _w4(_pad2(W_p1, 256, 256))
    wp2 = _w4(_pad2(W_p2, 256, 384))
    wa1 = _w4(_pad2(W_a1, 256, 256))
    wa2 = _w4(_pad2(W_a2, 256, 384))
    wp3 = jnp.zeros((768, 640), _f32)
    wp3 = wp3.at[:D2, :D4].set(W_p3[:D2]).at[384:384 + D2, :D4].set(W_p3[D2:])
    wp3 = _w4(wp3)
    bp1 = _pad2(b_p1[None, :], 1, 256).reshape(2, 1, F)
    bp2 = _pad2(b_p2[None, :], 1, 384).reshape(3, 1, F)
    ba1 = _pad2(b_a1[None, :], 1, 256).reshape(2, 1, F)
    ba2 = _pad2(b_a2[None, :], 1, 384).reshape(3, 1, F)
    bp3 = _pad2(b_p3[None, :], 1, 640).reshape(5, 1, F)
    wfc1 = jnp.zeros((1280, 1024), _f32)
    wfc1 = wfc1.at[:D4, :].set(W_fc1[:D4]).at[640:640 + D4, :].set(W_fc1[D4:])
    z2 = jnp.zeros((NP, F), _f32)

    degp, degr, counts = _sc_deg(sep, ser, batch2r, erows, z2)
    cnt = counts[:NB, :1]

    feat2d, dinvp, dinvr = _tc_prep(xpad, wbig, bbig,
                                    degp[:, :1], degr[:, :1])
    feat = feat2d.reshape(NP, 2, F).transpose(1, 0, 2)

    xw1 = _tc_mm(feat, wp1, dinvp)
    acc1 = _sc_conv(2, 0, xw1, sep, z2)
    x1 = _tc_ew(acc1, xw1, dinvp, bp1)
    xw2 = _tc_mm(x1, wp2, dinvp)
    acc2 = _sc_conv(3, 0, xw2, sep, z2)
    x2 = _tc_ew(acc2, xw2, dinvp, bp2)

    xw3 = _tc_mm(feat, wa1, dinvr)
    acc3 = _sc_conv(2, 1, xw3, ser, z2)
    y1 = _tc_ew(acc3, xw3, dinvr, ba1)
    xw4 = _tc_mm(y1, wa2, dinvr)
    acc4 = _sc_conv(3, 1, xw4, ser, z2)
    y2 = _tc_ew(acc4, xw4, dinvr, ba2)

    zc = jnp.concatenate([x2, y2], axis=0)
    xw5 = _tc_mm(zc, wp3, dinvp)
    acc5 = _sc_conv(5, 0, xw5, sep, z2)

    psum, pmax = _sc_pool(acc5, xw5, dinvp.reshape(NP), batch_pad, bp3, z2)
    zmean, zmax = _tc_comb(psum, pmax, cnt)
    zp = jnp.concatenate([zmean.transpose(1, 0, 2).reshape(NB, 640),
                          zmax.transpose(1, 0, 2).reshape(NB, 640)], axis=1)

    out = _tc_head(zp, wfc1, b_fc1.reshape(1, 1024), bn_gamma.reshape(1, 1024),
                   bn_beta.reshape(1, 1024), W_fc2, b_fc2.reshape(1, OUT_DIM))
    return out


# speedup vs baseline: 11.0672x; 1.0995x over previous
import functools

import jax
import jax.numpy as jnp
from jax import lax
from jax.experimental import pallas as pl
from jax.experimental.pallas import tpu as pltpu
from jax.experimental.pallas import tpu_sc as plsc

N = 10000
E = 640000
AA = 21
HID = 128
NB = 64
OUT_DIM = 486
D1 = HID + AA
D2 = D1 * 2
D4 = D1 * 4

NP = 10240
RPT = 640
F = 128
NT = 16
EB = 128
NBATCH = 320
G = 16
NW = NBATCH // G
EP = NT * NBATCH * EB
BQ = NP // EB

_f32 = jnp.float32
_i32 = jnp.int32

_MESH = plsc.VectorSubcoreMesh(core_axis_name="c", subcore_axis_name="s")


def _process_slab(tbl, acc, ebuf, rows, GS, SS):
    def sref(g):
        return ebuf.at[g].at[0].at[0]

    def dref(g):
        return ebuf.at[g].at[1].at[0]

    R = (rows.at[0], rows.at[1])
    pltpu.async_copy(tbl.at[sref(0)], R[0], GS[0])
    for g in range(G):
        A = g % 2
        B = 1 - A
        pltpu.make_async_copy(tbl.at[sref(g)], R[A], GS[A]).wait()
        if g < G - 1:
            if g >= 1:
                pltpu.make_async_copy(R[B], acc.at[dref(g - 1)], SS[B]).wait()
            pltpu.async_copy(tbl.at[sref(g + 1)], R[B], GS[B])
        pltpu.async_copy(R[A], acc.at[dref(g)], SS[A], add=True)
    pltpu.make_async_copy(R[0], acc.at[dref(G - 2)], SS[0]).wait()
    pltpu.make_async_copy(R[1], acc.at[dref(G - 1)], SS[1]).wait()


def _conv_pass(ch, xwc, se, z2, out, acc, eidx, rows, sid, r0, GS, SS, IS):
    pltpu.sync_copy(z2.at[pl.ds(r0, RPT)], acc.at[pl.ds(r0, RPT)])
    plsc.subcore_barrier()
    tbl = xwc.at[ch]
    sl = se.at[sid]
    pltpu.sync_copy(sl.at[0], eidx.at[0])

    def pair(p, carry):
        w = 2 * p
        pltpu.async_copy(sl.at[w + 1], eidx.at[1], IS[1])
        _process_slab(tbl, acc, eidx.at[0], rows, GS, SS)
        pltpu.make_async_copy(sl.at[w + 1], eidx.at[1], IS[1]).wait()

        @pl.when(p < NW // 2 - 1)
        def _():
            pltpu.async_copy(sl.at[w + 2], eidx.at[0], IS[0])

        _process_slab(tbl, acc, eidx.at[1], rows, GS, SS)

        @pl.when(p < NW // 2 - 1)
        def _():
            pltpu.make_async_copy(sl.at[w + 2], eidx.at[0], IS[0]).wait()

        return carry

    lax.fori_loop(0, NW // 2, pair, 0)
    plsc.subcore_barrier()
    pltpu.sync_copy(acc.at[pl.ds(r0, RPT)], out.at[ch].at[pl.ds(r0, RPT)])


_CONV_SCRATCH = [
    pltpu.VMEM_SHARED((NP, F), _f32),
    pltpu.VMEM((2, G, 2, 1, EB), _i32),
    pltpu.VMEM((2, EB, F), _f32),
    pltpu.SemaphoreType.DMA,
    pltpu.SemaphoreType.DMA,
    pltpu.SemaphoreType.DMA,
    pltpu.SemaphoreType.DMA,
    pltpu.SemaphoreType.DMA,
    pltpu.SemaphoreType.DMA,
]


def _sc_conv_ab(nce, xwc, sep, ser, z2):
    def body(xwc, sep, ser, z2, out, acc, eidx, rows,
             gs0, gs1, ss0, ss1, is0, is1):
        core = lax.axis_index("c")
        sid = lax.axis_index("s")
        r0 = sid * RPT
        GS, SS, IS = (gs0, gs1), (ss0, ss1), (is0, is1)
        for i in range(nce):
            @pl.when(core == 0)
            def _(i=i):
                _conv_pass(i, xwc, sep, z2, out, acc, eidx, rows, sid, r0,
                           GS, SS, IS)

            @pl.when(core == 1)
            def _(i=i):
                _conv_pass(nce + i, xwc, ser, z2, out, acc, eidx, rows,
                           sid, r0, GS, SS, IS)

    return pl.kernel(
        body,
        out_type=jax.ShapeDtypeStruct((2 * nce, NP, F), _f32),
        mesh=_MESH,
        scratch_types=_CONV_SCRATCH,
    )(xwc, sep, ser, z2)


def _sc_conv(nc, parity, xwc, se, z2):
    def body(xwc, se, z2, out, acc, eidx, rows, gs0, gs1, ss0, ss1, is0, is1):
        core = lax.axis_index("c")
        sid = lax.axis_index("s")
        r0 = sid * RPT
        GS, SS, IS = (gs0, gs1), (ss0, ss1), (is0, is1)
        for ch in range(nc):
            @pl.when(core == (ch + parity) % 2)
            def _(ch=ch):
                _conv_pass(ch, xwc, se, z2, out, acc, eidx, rows, sid, r0,
                           GS, SS, IS)

    return pl.kernel(
        body,
        out_type=jax.ShapeDtypeStruct((nc, NP, F), _f32),
        mesh=_MESH,
        scratch_types=_CONV_SCRATCH,
    )(xwc, se, z2)


def _deg_body(sep, ser, batch2r, erows, z2, degp, degr, counts,
              dacc, cacc, eidx, bidx, eones, ss0):
    core = lax.axis_index("c")
    sid = lax.axis_index("s")
    r0 = sid * RPT
    pltpu.sync_copy(erows, eones)
    pltpu.sync_copy(z2.at[pl.ds(r0, RPT)], dacc.at[pl.ds(r0, RPT)])
    pltpu.sync_copy(z2.at[pl.ds(sid * 8, 8)], cacc.at[pl.ds(sid * 8, 8)])
    plsc.subcore_barrier()

    def wloop(w, carry):
        @pl.when(core == 0)
        def _():
            pltpu.sync_copy(sep.at[sid].at[w], eidx)

        @pl.when(core == 1)
        def _():
            pltpu.sync_copy(ser.at[sid].at[w], eidx)

        for g in range(G):
            pltpu.async_copy(eones, dacc.at[eidx.at[g].at[1].at[0]], ss0,
                             add=True)
        for g in range(G):
            pltpu.make_async_copy(eones, dacc.at[eidx.at[g].at[1].at[0]],
                                  ss0).wait()
        return carry

    lax.fori_loop(0, NW, wloop, 0)

    @pl.when(core == 0)
    def _():

        def cb(b, carry):
            pltpu.sync_copy(batch2r.at[sid].at[b].at[0], bidx)
            pltpu.sync_copy(eones, cacc.at[bidx], add=True)
            return carry

        lax.fori_loop(0, BQ // NT, cb, 0)

    plsc.subcore_barrier()

    @pl.when(core == 0)
    def _():
        pltpu.sync_copy(dacc.at[pl.ds(r0, RPT)], degp.at[pl.ds(r0, RPT)])
        pltpu.sync_copy(cacc.at[pl.ds(sid * 8, 8)],
                        counts.at[pl.ds(sid * 8, 8)])

    @pl.when(core == 1)
    def _():
        pltpu.sync_copy(dacc.at[pl.ds(r0, RPT)], degr.at[pl.ds(r0, RPT)])


def _sc_deg(sep, ser, batch2r, erows, z2):
    return pl.kernel(
        _deg_body,
        out_type=(
            jax.ShapeDtypeStruct((NP, F), _f32),
            jax.ShapeDtypeStruct((NP, F), _f32),
            jax.ShapeDtypeStruct((EB, F), _f32),
        ),
        mesh=_MESH,
        scratch_types=[
            pltpu.VMEM_SHARED((NP, F), _f32),
            pltpu.VMEM_SHARED((EB, F), _f32),
            pltpu.VMEM((G, 2, 1, EB), _i32),
            pltpu.VMEM((EB,), _i32),
            pltpu.VMEM((EB, F), _f32),
            pltpu.SemaphoreType.DMA,
        ],
    )(sep, ser, batch2r, erows, z2)


def _pool_body(acc5, xw5, dinv, batchp, b3p, z2, psum, pmax,
               abuf, xbuf, sacc, macc, bvec, dbuf, bibuf):
    core = lax.axis_index("c")
    sid = lax.axis_index("s")
    r0 = sid * RPT
    pltpu.sync_copy(dinv.at[pl.ds(r0, RPT)], dbuf)
    pltpu.sync_copy(batchp.at[pl.ds(r0, RPT)], bibuf)

    for ch in range(5):
        @pl.when(core == ch % 2)
        def _(ch=ch):
            pltpu.sync_copy(b3p.at[ch], bvec)
            pltpu.sync_copy(z2.at[pl.ds(0, BQ)], sacc)
            pltpu.sync_copy(z2.at[pl.ds(0, BQ)], macc)

            def blk_body(blk, carry2):
                pltpu.sync_copy(acc5.at[ch].at[pl.ds(r0 + blk * EB, EB)], abuf)
                pltpu.sync_copy(xw5.at[ch].at[pl.ds(r0 + blk * EB, EB)], xbuf)

                def grpbody(g, carry3):
                    b16 = bibuf[pl.ds(blk * EB + g * 16, 16)]
                    d16 = dbuf[pl.ds(blk * EB + g * 16, 16)]
                    for k in range(16):
                        i = g * 16 + k
                        b = b16[k]
                        dv = d16[k]
                        for j in range(F // 16):
                            sl = pl.ds(j * 16, 16)
                            v = (abuf[i, sl] + xbuf[i, sl]) * dv + bvec[0, sl]
                            v = jnp.maximum(v, 0.0)
                            sacc[b, sl] = sacc[b, sl] + v
                            macc[b, sl] = jnp.maximum(macc[b, sl], v)
                    return carry3

                lax.fori_loop(0, EB // 16, grpbody, 0)
                return carry2

            lax.fori_loop(0, RPT // EB, blk_body, 0)
            pltpu.sync_copy(sacc.at[pl.ds(0, NB)], psum.at[ch].at[sid])
            pltpu.sync_copy(macc.at[pl.ds(0, NB)], pmax.at[ch].at[sid])


def _sc_pool(acc5, xw5, dinv, batchp, b3p, z2):
    return pl.kernel(
        _pool_body,
        out_type=(
            jax.ShapeDtypeStruct((5, NT, NB, F), _f32),
            jax.ShapeDtypeStruct((5, NT, NB, F), _f32),
        ),
        mesh=_MESH,
        scratch_types=[
            pltpu.VMEM((EB, F), _f32),
            pltpu.VMEM((EB, F), _f32),
            pltpu.VMEM((BQ, F), _f32),
            pltpu.VMEM((BQ, F), _f32),
            pltpu.VMEM((1, 128), _f32),
            pltpu.VMEM((RPT,), _f32),
            pltpu.VMEM((RPT,), _i32),
        ],
    )(acc5, xw5, dinv, batchp, b3p, z2)


def _prep_body(x_ref, wbig_ref, bbig_ref, degp_ref, degr_ref,
               feat_ref, dinvp_ref, dinvr_ref):
    feat_ref[...] = jnp.maximum(
        jnp.dot(x_ref[...], wbig_ref[...], preferred_element_type=_f32)
        + bbig_ref[...], 0.0)
    mask = (lax.broadcasted_iota(_i32, (NP, 1), 0) < N).astype(_f32)
    dinvp_ref[...] = lax.rsqrt(degp_ref[...] + 1.0) * mask
    dinvr_ref[...] = lax.rsqrt(degr_ref[...] + 1.0) * mask


def _tc_prep(xpad, wbig, bbig, degp, degr):
    return pl.pallas_call(
        _prep_body,
        out_shape=(
            jax.ShapeDtypeStruct((NP, 256), _f32),
            jax.ShapeDtypeStruct((NP, 1), _f32),
            jax.ShapeDtypeStruct((NP, 1), _f32),
        ),
    )(xpad, wbig, bbig, degp, degr)


def _mm_body(nci, x_ref, w_ref, dout_ref, out_ref):
    ci = pl.program_id(1)
    h = jnp.dot(x_ref[0], w_ref[0, 0], preferred_element_type=_f32)

    @pl.when(ci == 0)
    def _():
        out_ref[0] = h

    @pl.when(ci > 0)
    def _():
        out_ref[0] = out_ref[0] + h

    @pl.when(ci == nci - 1)
    def _():
        out_ref[0] = dout_ref[0] * out_ref[0]


def _tc_mm(x3, w4, dinv2, so):
    nci, nco = w4.shape[0], w4.shape[1]
    return pl.pallas_call(
        functools.partial(_mm_body, nci),
        grid=(nco, nci),
        in_specs=[
            pl.BlockSpec((1, NP, F), lambda c, ci: (ci, 0, 0)),
            pl.BlockSpec((1, 1, F, F), lambda c, ci: (ci, c, 0, 0)),
            pl.BlockSpec((1, NP, 1), lambda c, ci, so=so: (c // so, 0, 0)),
        ],
        out_specs=pl.BlockSpec((1, NP, F), lambda c, ci: (c, 0, 0)),
        out_shape=jax.ShapeDtypeStruct((nco, NP, F), _f32),
    )(x3, w4, dinv2)


def _mme_body(nci, acc_ref, xw_ref, din_ref, b_ref, w_ref, dout_ref, out_ref):
    ci = pl.program_id(1)
    xin = jnp.maximum(
        din_ref[0] * (acc_ref[0] + xw_ref[0]) + b_ref[0], 0.0)
    h = jnp.dot(xin, w_ref[0, 0], preferred_element_type=_f32)

    @pl.when(ci == 0)
    def _():
        out_ref[0] = h

    @pl.when(ci > 0)
    def _():
        out_ref[0] = out_ref[0] + h

    @pl.when(ci == nci - 1)
    def _():
        out_ref[0] = dout_ref[0] * out_ref[0]


def _tc_mme(acc, xw, dinv2_in, bpad, w4, dinv2_out, si, so):
    nci, nco = w4.shape[0], w4.shape[1]
    return pl.pallas_call(
        functools.partial(_mme_body, nci),
        grid=(nco, nci),
        in_specs=[
            pl.BlockSpec((1, NP, F), lambda c, ci: (ci, 0, 0)),
            pl.BlockSpec((1, NP, F), lambda c, ci: (ci, 0, 0)),
            pl.BlockSpec((1, NP, 1), lambda c, ci, si=si: (ci // si, 0, 0)),
            pl.BlockSpec((1, 1, F), lambda c, ci: (ci, 0, 0)),
            pl.BlockSpec((1, 1, F, F), lambda c, ci: (ci, c, 0, 0)),
            pl.BlockSpec((1, NP, 1), lambda c, ci, so=so: (c // so, 0, 0)),
        ],
        out_specs=pl.BlockSpec((1, NP, F), lambda c, ci: (c, 0, 0)),
        out_shape=jax.ShapeDtypeStruct((nco, NP, F), _f32),
    )(acc, xw, dinv2_in, bpad, w4, dinv2_out)


def _comb_body(psum_ref, pmax_ref, cnt_ref, zmean_ref, zmax_ref):
    denom = jnp.maximum(cnt_ref[...], 1.0)
    zmean_ref[0] = jnp.sum(psum_ref[0], axis=0) / denom
    zmax_ref[0] = jnp.max(pmax_ref[0], axis=0)


def _tc_comb(psum, pmax, cnt):
    return pl.pallas_call(
        _comb_body,
        grid=(5,),
        in_specs=[
            pl.BlockSpec((1, NT, NB, F), lambda c: (c, 0, 0, 0)),
            pl.BlockSpec((1, NT, NB, F), lambda c: (c, 0, 0, 0)),
            pl.BlockSpec((NB, 1), lambda c: (0, 0)),
        ],
        out_specs=(
            pl.BlockSpec((1, NB, F), lambda c: (c, 0, 0)),
            pl.BlockSpec((1, NB, F), lambda c: (c, 0, 0)),
        ),
        out_shape=(
            jax.ShapeDtypeStruct((5, NB, F), _f32),
            jax.ShapeDtypeStruct((5, NB, F), _f32),
        ),
    )(psum, pmax, cnt)


def _head_body(zp_ref, wfc1_ref, bfc1_ref, gamma_ref, beta_ref, wfc2_ref,
               bfc2_ref, out_ref):
    h = jnp.dot(zp_ref[...], wfc1_ref[...], preferred_element_type=_f32) \
        + bfc1_ref[...]
    mu = jnp.mean(h, axis=0, keepdims=True)
    var = jnp.mean((h - mu) ** 2, axis=0, keepdims=True)
    h = (h - mu) * lax.rsqrt(var + 1e-5) * gamma_ref[...] + beta_ref[...]
    h = jnp.maximum(h, 0.0)
    o = jnp.dot(h, wfc2_ref[...], preferred_element_type=_f32) + bfc2_ref[...]
    out_ref[...] = jax.nn.sigmoid(o)


def _tc_head(zp, wfc1, bfc1, gamma, beta, wfc2, bfc2):
    return pl.pallas_call(
        _head_body,
        out_shape=jax.ShapeDtypeStruct((NB, OUT_DIM), _f32),
    )(zp, wfc1, bfc1, gamma, beta, wfc2, bfc2)


def _pad2(w, rows, cols):
    return jnp.zeros((rows, cols), _f32).at[:w.shape[0], :w.shape[1]].set(w)


def _w4(w2d):
    kp, dp = w2d.shape
    return w2d.reshape(kp // F, F, dp // F, F).transpose(0, 2, 1, 3)


def _pad_edges(ei):
    npad = EP - E
    psrc = (jnp.arange(npad, dtype=_i32) % 8192)
    pdst = N + (jnp.arange(npad, dtype=_i32) % (NP - N))
    src = jnp.concatenate([ei[0], psrc]).reshape(NT, NBATCH, EB)
    dst = jnp.concatenate([ei[1], pdst]).reshape(NT, NBATCH, EB)
    se = jnp.stack([src, dst], axis=2)
    return se.reshape(NT, NW, G, 2, 1, EB)


def kernel(prot_x, prot_edge_index, prot_edge_index_replace, prot_batch,
           W_fl1, b_fl1, W_fl2, b_fl2, W_p1, b_p1, W_p2, b_p2, W_a1, b_a1,
           W_a2, b_a2, W_p3, b_p3, W_fc1, b_fc1, bn_gamma, bn_beta,
           W_fc2, b_fc2):
    sep = _pad_edges(prot_edge_index)
    ser = _pad_edges(prot_edge_index_replace)
    batch_pad = jnp.concatenate(
        [prot_batch, NB + (jnp.arange(NP - N, dtype=_i32) % (BQ - NB))])
    batch2r = batch_pad.reshape(NT, BQ // NT, 1, EB)
    erows = jnp.zeros((EB, F), _f32).at[:, 0].set(1.0)
    xpad = _pad2(prot_x, NP, 256)
    wbig = jnp.zeros((256, 256), _f32)
    wbig = wbig.at[:AA, :AA].set(W_fl2).at[AA:D1, AA:D1].set(W_fl1)
    bbig = jnp.zeros((1, 256), _f32).at[0, :AA].set(b_fl2).at[0, AA:D1].set(b_fl1)
    w13 = jnp.zeros((2, 4, F, F), _f32)
    w13 = w13.at[:, :2].set(_w4(_pad2(W_p1, 256, 256)))
    w13 = w13.at[:, 2:].set(_w4(_pad2(W_a1, 256, 256)))
    w24 = jnp.zeros((4, 6, F, F), _f32)
    w24 = w24.at[:2, :3].set(_w4(_pad2(W_p2, 256, 384)))
    w24 = w24.at[2:, 3:].set(_w4(_pad2(W_a2, 256, 384)))
    wp3 = jnp.zeros((768, 640), _f32)
    wp3 = wp3.at[:D2, :D4].set(W_p3[:D2]).at[384:384 + D2, :D4].set(W_p3[D2:])
    w5 = _w4(wp3)
    bp1 = _pad2(b_p1[None, :], 1, 256).reshape(2, 1, F)
    bp2 = _pad2(b_p2[None, :], 1, 384).reshape(3, 1, F)
    ba1 = _pad2(b_a1[None, :], 1, 256).reshape(2, 1, F)
    ba2 = _pad2(b_a2[None, :], 1, 384).reshape(3, 1, F)
    b13 = jnp.concatenate([bp1, ba1], axis=0)
    b24 = jnp.concatenate([bp2, ba2], axis=0)
    bp3 = _pad2(b_p3[None, :], 1, 640).reshape(5, 1, F)
    wfc1 = jnp.zeros((1280, 1024), _f32)
    wfc1 = wfc1.at[:D4, :].set(W_fc1[:D4]).at[640:640 + D4, :].set(W_fc1[D4:])
    z2 = jnp.zeros((NP, F), _f32)

    degp, degr, counts = _sc_deg(sep, ser, batch2r, erows, z2)
    cnt = counts[:NB, :1]

    feat2d, dinvp, dinvr = _tc_prep(xpad, wbig, bbig,
                                    degp[:, :1], degr[:, :1])
    feat = feat2d.reshape(NP, 2, F).transpose(1, 0, 2)
    dinv2 = jnp.stack([dinvp, dinvr], axis=0)

    xw13 = _tc_mm(feat, w13, dinv2, 2)
    acc13 = _sc_conv_ab(2, xw13, sep, ser, z2)
    xw24 = _tc_mme(acc13, xw13, dinv2, b13, w24, dinv2, 2, 3)
    acc24 = _sc_conv_ab(3, xw24, sep, ser, z2)
    xw5 = _tc_mme(acc24, xw24, dinv2, b24, w5, dinv2[:1], 3, 5)
    acc5 = _sc_conv(5, 0, xw5, sep, z2)

    psum, pmax = _sc_pool(acc5, xw5, dinvp.reshape(NP), batch_pad, bp3, z2)
    zmean, zmax = _tc_comb(psum, pmax, cnt)
    zp = jnp.concatenate([zmean.transpose(1, 0, 2).reshape(NB, 640),
                          zmax.transpose(1, 0, 2).reshape(NB, 640)], axis=1)

    out = _tc_head(zp, wfc1, b_fc1.reshape(1, 1024), bn_gamma.reshape(1, 1024),
                   bn_beta.reshape(1, 1024), W_fc2, b_fc2.reshape(1, OUT_DIM))
    return out
